# Initial kernel scaffold; baseline (speedup 1.0000x reference)
import jax
import jax.numpy as jnp
from jax.experimental import pallas as pl


def kernel(edge_index, edge_type, rel_query, h_index, t_index, r_index, W, b, Wm1, bm1, Wm2, bm2):
    raise NotImplementedError("write your pallas kernel here")


# speedup vs baseline: 1552.8056x; 1552.8056x over previous
import jax
import jax.numpy as jnp
from jax.experimental import pallas as pl
from jax.experimental.pallas import tpu as pltpu

N_NODES = 10000
NUM_REL = 16
DIM = 128
E_TOTAL = 160000
BS = 2
NEG = 8
P = BS * NEG

BLK = 16000
G = E_TOTAL // BLK

_NT = (((1,), (1,)), ((), ()))


def _body(src_ref, dst_ref, et_ref, hv_ref, tv_ref, rv_ref,
          rq_ref, w_ref, b_ref, wm1_ref, bm1_ref, wm2_ref, bm2_ref,
          out_ref, cf_ref, ci_ref):
    i = pl.program_id(0)

    @pl.when(i == 0)
    def _init():
        cf_ref[...] = jnp.zeros_like(cf_ref)
        ci_ref[...] = jnp.zeros_like(ci_ref)

    s = src_ref[0]
    d = dst_ref[0]
    t = et_ref[0]
    hv = hv_ref[...]
    tv = tv_ref[...]

    a_f = jnp.logical_and(s == hv, d == tv).astype(jnp.bfloat16)
    a_i = jnp.logical_and(d == hv, s == tv).astype(jnp.bfloat16)
    rel_iota = jax.lax.broadcasted_iota(jnp.int32, (NUM_REL, 1), 0)
    oh = (t == rel_iota).astype(jnp.bfloat16)

    cf_ref[...] += jax.lax.dot_general(oh, a_f, _NT, preferred_element_type=jnp.float32)
    ci_ref[...] += jax.lax.dot_general(oh, a_i, _NT, preferred_element_type=jnp.float32)

    @pl.when(i == G - 1)
    def _finish():
        ii = jax.lax.broadcasted_iota(jnp.int32, (P, P), 0)
        jj = jax.lax.broadcasted_iota(jnp.int32, (P, P), 1)
        eye = (ii == jj).astype(jnp.float32)
        cnt_f = jax.lax.dot_general(eye, cf_ref[...], _NT,
                                    preferred_element_type=jnp.float32)
        cnt_i = jax.lax.dot_general(eye, ci_ref[...], _NT,
                                    preferred_element_type=jnp.float32)
        rq = rq_ref[...]
        S = cnt_f @ rq[:NUM_REL] + cnt_i @ rq[NUM_REL:]

        r_iota = jax.lax.broadcasted_iota(jnp.int32, (P, 2 * NUM_REL), 1)
        oh_r = (rv_ref[...] == r_iota).astype(jnp.float32)
        q = oh_r @ rq

        is_head = (tv_ref[...] == hv_ref[...]).astype(jnp.float32)
        agg = q * (S + is_head)
        hin = is_head * q
        h1 = jnp.maximum(hin @ w_ref[:DIM] + agg @ w_ref[DIM:] + b_ref[...], 0.0)
        hm = jnp.maximum(h1 @ wm1_ref[:DIM] + q @ wm1_ref[DIM:] + bm1_ref[...], 0.0)
        out_ref[...] = hm @ wm2_ref[...] + bm2_ref[...]


def kernel(edge_index, edge_type, rel_query, h_index, t_index, r_index,
           W, b, Wm1, bm1, Wm2, bm2):
    src = edge_index[0].reshape(G, 1, BLK)
    dst = edge_index[1].reshape(G, 1, BLK)
    et = edge_type.reshape(G, 1, BLK)
    hv = h_index.reshape(P, 1)
    tv = t_index.reshape(P, 1)
    rv = r_index.reshape(P, 1)

    edge_spec = pl.BlockSpec((1, 1, BLK), lambda i: (i, 0, 0))
    whole = lambda shape: pl.BlockSpec(shape, lambda i: tuple(0 for _ in shape))

    out = pl.pallas_call(
        _body,
        grid=(G,),
        in_specs=[
            edge_spec, edge_spec, edge_spec,
            whole((P, 1)), whole((P, 1)), whole((P, 1)),
            whole((2 * NUM_REL, DIM)),
            whole((2 * DIM, DIM)), whole((1, DIM)),
            whole((2 * DIM, 2 * DIM)), whole((1, 2 * DIM)),
            whole((2 * DIM, 1)), whole((1, 1)),
        ],
        out_specs=pl.BlockSpec((P, 1), lambda i: (0, 0)),
        out_shape=jax.ShapeDtypeStruct((P, 1), jnp.float32),
        scratch_shapes=[
            pltpu.VMEM((NUM_REL, P), jnp.float32),
            pltpu.VMEM((NUM_REL, P), jnp.float32),
        ],
        compiler_params=pltpu.CompilerParams(
            dimension_semantics=("arbitrary",),
        ),
    )(src, dst, et, hv, tv, rv, rel_query,
      W, b.reshape(1, DIM), Wm1, bm1.reshape(1, 2 * DIM), Wm2, bm2.reshape(1, 1))
    return out[:, 0].reshape(BS, NEG)
